# Initial kernel scaffold; baseline (speedup 1.0000x reference)
import jax
import jax.numpy as jnp
from jax.experimental import pallas as pl


def kernel(h, h_full, z, mask, e_feat, att_src, att_dst, att_dist, att_vec, z_emb_table, Wvw0, bvw0, Wvw1, bvw1, Wg0, bg0, Wg1, bg1, We0, be0, We1, be1, Wo0, bo0, Wo1, bo1, Wo2, bo2):
    raise NotImplementedError("write your pallas kernel here")


# speedup vs baseline: 4.9467x; 4.9467x over previous
import functools
import math

import numpy as np
import jax
import jax.numpy as jnp
from jax import lax
from jax.experimental import pallas as pl
from jax.experimental.pallas import tpu as pltpu
from jax.experimental.pallas import tpu_sc as plsc

_CUT = 5.0
_NRBF = 16
_SQ2 = math.sqrt(2.0)
_SQ3 = math.sqrt(3.0)
_SQ8 = math.sqrt(8.0)

_NC = 2
_NS = 16
_NW = _NC * _NS
_CHUNK = 128


def _fixed_mats():
    R1 = np.zeros((16, 256), np.float32)
    Q1 = np.zeros((256, 16), np.float32)
    for i in range(16):
        for k in range(16):
            R1[i, i * 16 + k] = 1.0
            Q1[i * 16 + k, k] = 1.0
    R2 = np.zeros((16, 128), np.float32)
    Q2 = np.zeros((128, 8), np.float32)
    for i in range(16):
        for k in range(8):
            R2[i, i * 8 + k] = 1.0
            Q2[i * 8 + k, k] = 1.0
    Ra = np.zeros((24, 192), np.float32)
    Rb = np.zeros((64, 192), np.float32)
    Qv = np.zeros((192, 24), np.float32)
    for i in range(8):
        for k in range(8):
            for c in range(3):
                Ra[i * 3 + c, i * 24 + k * 3 + c] = 1.0
                Rb[i * 8 + k, i * 24 + k * 3 + c] = 1.0
                Qv[i * 24 + k * 3 + c, k * 3 + c] = 1.0
    Rs = np.zeros((3, 24), np.float32)
    for m in range(8):
        for c in range(3):
            Rs[c, m * 3 + c] = 1.0
    Qd = np.zeros((24, 8), np.float32)
    for j in range(8):
        for c in range(3):
            Qd[j * 3 + c, j] = 1.0
    Rd = np.zeros((8, 128), np.float32)
    Q4 = np.zeros((128, 16), np.float32)
    for i in range(8):
        for k in range(16):
            Rd[i, i * 16 + k] = 1.0
            Q4[i * 16 + k, k] = 1.0
    Rt = np.zeros((8, 24), np.float32)
    for k in range(8):
        for c in range(3):
            Rt[k, k * 3 + c] = 1.0
    Rsf = np.zeros((24, 40), np.float32)
    for k in range(16):
        Rsf[k, k] = 1.0
    for j in range(8):
        for c in range(3):
            Rsf[16 + j, 16 + j * 3 + c] = 1.0
    return dict(R1=R1, Q1=Q1, R2=R2, Q2=Q2, Ra=Ra, Rb=Rb, Qv=Qv, Rs=Rs,
                Qd=Qd, Rd=Rd, Q4=Q4, Rt=Rt, Rsf=Rsf)


_MATS = _fixed_mats()


def _silu(x):
    return x * jax.nn.sigmoid(x)


def _prep_body(h_ref, hf_ref, zcol_ref, tab_ref, out_ref):
    bp = h_ref.shape[0]
    zcol = zcol_ref[...]
    iota = lax.broadcasted_iota(jnp.int32, (bp, 128), 1).astype(jnp.float32)
    oh = (zcol == iota).astype(jnp.float32)
    zr = jnp.dot(oh, tab_ref[...], preferred_element_type=jnp.float32)
    pad = jnp.zeros((bp, 56), jnp.float32)
    out_ref[...] = jnp.concatenate([h_ref[...], zr, hf_ref[...], pad], axis=1)


def _prep_call(h2, hf2, zcol, tab_pad, bp):
    n = h2.shape[0]
    return pl.pallas_call(
        _prep_body,
        grid=(n // bp,),
        in_specs=[
            pl.BlockSpec((bp, 128), lambda i: (i, 0)),
            pl.BlockSpec((bp, 40), lambda i: (i, 0)),
            pl.BlockSpec((bp, 1), lambda i: (i, 0)),
            pl.BlockSpec((128, 32), lambda i: (0, 0)),
        ],
        out_specs=pl.BlockSpec((bp, 256), lambda i: (i, 0)),
        out_shape=jax.ShapeDtypeStruct((n, 256), jnp.float32),
    )(h2, hf2, zcol, tab_pad)


def _gather_call(tdst, hsrc, idx_dst, idx_src):
    e = idx_dst.shape[0]
    nchunks = e // _CHUNK
    iters = (nchunks + _NW - 1) // _NW
    mesh = plsc.VectorSubcoreMesh(core_axis_name="c", subcore_axis_name="s")

    @functools.partial(
        pl.kernel,
        out_type=[jax.ShapeDtypeStruct((e, 256), jnp.float32),
                  jax.ShapeDtypeStruct((e, 128), jnp.float32)],
        mesh=mesh,
        scratch_types=[
            pltpu.VMEM((_CHUNK,), jnp.int32),
            pltpu.VMEM((_CHUNK,), jnp.int32),
            pltpu.VMEM((_CHUNK, 256), jnp.float32),
            pltpu.VMEM((_CHUNK, 128), jnp.float32),
            pltpu.SemaphoreType.DMA,
            pltpu.SemaphoreType.DMA,
        ],
    )
    def gather_k(tdst_hbm, hsrc_hbm, idxd_hbm, idxs_hbm, outd_hbm, outs_hbm,
                 idxd_v, idxs_v, rowsd_v, rowss_v, semd, sems):
        wid = lax.axis_index("s") * _NC + lax.axis_index("c")

        def body(j, carry):
            cid = j * _NW + wid

            @pl.when(cid < nchunks)
            def _():
                base = cid * _CHUNK
                pltpu.sync_copy(idxd_hbm.at[pl.ds(base, _CHUNK)], idxd_v)
                pltpu.sync_copy(idxs_hbm.at[pl.ds(base, _CHUNK)], idxs_v)
                cpd = pltpu.make_async_copy(tdst_hbm.at[idxd_v], rowsd_v, semd)
                cps = pltpu.make_async_copy(hsrc_hbm.at[idxs_v], rowss_v, sems)
                cpd.start()
                cps.start()
                cpd.wait()
                cps.wait()
                pltpu.sync_copy(rowsd_v, outd_hbm.at[pl.ds(base, _CHUNK)])
                pltpu.sync_copy(rowss_v, outs_hbm.at[pl.ds(base, _CHUNK)])

            return carry

        lax.fori_loop(0, iters, body, 0)

    return gather_k(tdst, hsrc, idx_dst, idx_src)


def _edge_body(dstg, srcg, esc,
               Wvw0a, wv0self, Wvw0c, bvw0, Wvw1, bvw1,
               Wg0a, Wg0b, Wg0c, wg0self, bg0, Wg1, bg1,
               R1, Q1, R2, Q2, Ra, Rb, Qv, Rs, Qd, Rd, Q4, Rt,
               out_ref):
    f32 = jnp.float32
    d = esc[:, 0:1]
    vvec = esc[:, 1:4]
    is_self = (esc[:, 4:5] == esc[:, 5:6]).astype(f32)
    eps = jnp.maximum(d, 1e-8)
    u = vvec / eps
    un = jnp.sqrt(jnp.sum(u * u, axis=1, keepdims=True))
    u_sh = u / jnp.maximum(un, 1e-8)
    sh1 = _SQ3 * u_sh * (1.0 - is_self)

    width = _CUT / (_NRBF - 1)
    centers = lax.broadcasted_iota(jnp.int32, (1, _NRBF), 1).astype(f32) * width
    rbf = jnp.exp(-0.5 * ((d - centers) / width) ** 2)

    hdst = dstg[:, 0:128]
    zr = dstg[:, 128:160]
    s1 = dstg[:, 160:176]
    v1f = dstg[:, 176:200]

    def mm(a, b):
        return jnp.dot(a, b[...], preferred_element_type=f32)

    wh = _silu(mm(zr, Wvw0a) + mm(rbf, Wvw0c) + is_self * wv0self[...] + bvw0[...])
    tpw = mm(wh, Wvw1) + bvw1[...]
    tpw1 = tpw[:, 0:256]
    tpw2 = tpw[:, 256:384]
    tpw3 = tpw[:, 384:448]
    tpw4 = tpw[:, 448:576]

    gh = _silu(mm(srcg[...], Wg0a) + mm(hdst, Wg0b) + mm(rbf, Wg0c)
               + is_self * wg0self[...] + bg0[...])
    gate = jax.nn.sigmoid(mm(gh, Wg1) + bg1[...])

    os_a = mm(mm(s1, R1) * tpw1, Q1) * 0.25
    t2 = mm(mm(s1, R2) * tpw2, Q2)
    sh1r = mm(sh1, Rs)
    ov_a = mm(t2, Rt) * sh1r * (1.0 / (4.0 * _SQ3))
    ov_b = mm(mm(v1f, Ra) * mm(tpw3, Rb), Qv) * (1.0 / _SQ8)
    dots = mm(v1f * sh1r, Qd) * (1.0 / _SQ3)
    os_b = mm(mm(dots, Rd) * tpw4, Q4) * (1.0 / _SQ8)

    os = (os_a + os_b) * (1.0 / _SQ2)
    ov = (ov_a + ov_b) * (1.0 / _SQ2)

    env = 0.5 * (jnp.cos(jnp.pi * d / _CUT) + 1.0) * (d < _CUT).astype(f32)
    g = env * gate
    zpad = jnp.zeros((os.shape[0], 88), f32)
    out_ref[...] = jnp.concatenate([os * g, ov * g, zpad], axis=1)


def _edge_call(dstg, srcg, esc, wts, mats, bk):
    e = dstg.shape[0]
    full = lambda shp: pl.BlockSpec(shp, lambda i: tuple(0 for _ in shp))
    consts = [wts["Wvw0a"], wts["wv0self"], wts["Wvw0c"], wts["bvw0"],
              wts["Wvw1"], wts["bvw1"], wts["Wg0a"], wts["Wg0b"], wts["Wg0c"],
              wts["wg0self"], wts["bg0"], wts["Wg1"], wts["bg1"],
              mats["R1"], mats["Q1"], mats["R2"], mats["Q2"], mats["Ra"],
              mats["Rb"], mats["Qv"], mats["Rs"], mats["Qd"], mats["Rd"],
              mats["Q4"], mats["Rt"]]
    return pl.pallas_call(
        _edge_body,
        grid=(e // bk,),
        in_specs=[
            pl.BlockSpec((bk, 256), lambda i: (i, 0)),
            pl.BlockSpec((bk, 128), lambda i: (i, 0)),
            pl.BlockSpec((bk, 8), lambda i: (i, 0)),
        ] + [full(c.shape) for c in consts],
        out_specs=pl.BlockSpec((bk, 128), lambda i: (i, 0)),
        out_shape=jax.ShapeDtypeStruct((e, 128), jnp.float32),
    )(dstg, srcg, esc, *consts)


_RNG = 2560
_RJ = 2688


def _scatter_call(ve, idx_src, zeros_init, npad):
    e = ve.shape[0]
    nchunks = e // _CHUNK
    iters = (nchunks + _NS - 1) // _NS
    passes = npad // _RNG // _NC
    zrows = _RJ // _NS
    drows = _RNG // _NS
    mesh = plsc.VectorSubcoreMesh(core_axis_name="c", subcore_axis_name="s")

    @functools.partial(
        pl.kernel,
        out_type=jax.ShapeDtypeStruct((npad, 128), jnp.float32),
        mesh=mesh,
        scratch_types=[
            pltpu.VMEM((zrows, 128), jnp.float32),
            pltpu.VMEM((_CHUNK, 128), jnp.float32),
            pltpu.VMEM((_CHUNK,), jnp.int32),
            pltpu.VMEM_SHARED((_RJ, 128), jnp.float32),
        ],
    )
    def scatter_k(ve_hbm, idx_hbm, zeros_hbm, out_hbm,
                  stage_v, rows_v, idx_v, acc):
        core = lax.axis_index("c")
        tid = lax.axis_index("s")

        for p in range(passes):
            rng_id = core * passes + p
            base_node = rng_id * _RNG

            pltpu.sync_copy(zeros_hbm, stage_v)
            pltpu.sync_copy(stage_v, acc.at[pl.ds(tid * zrows, zrows)])
            plsc.subcore_barrier()

            def body(j, carry):
                cid = j * _NS + tid

                @pl.when(cid < nchunks)
                def _():
                    base = cid * _CHUNK
                    pltpu.sync_copy(ve_hbm.at[pl.ds(base, _CHUNK)], rows_v)
                    pltpu.sync_copy(idx_hbm.at[pl.ds(base, _CHUNK)], idx_v)
                    for i in range(_CHUNK // 16):
                        v = idx_v[pl.ds(i * 16, 16)] - base_node
                        ok = (v >= 0) & (v < _RNG)
                        idx_v[pl.ds(i * 16, 16)] = jnp.where(ok, v, _RNG)
                    pltpu.sync_copy(rows_v, acc.at[idx_v], add=True)

                return carry

            lax.fori_loop(0, iters, body, 0)
            plsc.subcore_barrier()

            pltpu.sync_copy(acc.at[pl.ds(tid * drows, drows)],
                            stage_v.at[pl.ds(0, drows)])
            pltpu.sync_copy(stage_v.at[pl.ds(0, drows)],
                            out_hbm.at[pl.ds(base_node + tid * drows, drows)])
            plsc.subcore_barrier()

    return scatter_k(ve, idx_src, zeros_init)


def _node_body(parts, efeat, We0, be0, We1, be1, Rsf, Qd,
               Wo0, bo0, Wo1, bo1, Wo2, bo2, out_ref):
    f32 = jnp.float32

    def mm(a, b):
        return jnp.dot(a, b[...], preferred_element_type=f32)

    oi = parts[:, 0:40]
    sc0 = mm(_silu(mm(efeat[...], We0) + be0[...]), We1) + be1[...]
    sf = mm(sc0, Rsf)
    for t in range(8):
        row = sf[t:t + 1, :]
        vm = oi * row
        sp = vm[:, 0:16]
        vv = vm[:, 16:40]
        n2 = mm(vv * vv, Qd) + 1e-8
        inv = jnp.concatenate([sp, jnp.sqrt(n2)], axis=1)
        x = _silu(mm(inv, Wo0) + bo0[...])
        x = _silu(mm(x, Wo1) + bo1[...])
        out_ref[:, t, :] = mm(x, Wo2) + bo2[...]


def _node_call(parts, efeat, wts, mats, n, bn):
    full = lambda shp: pl.BlockSpec(shp, lambda i: tuple(0 for _ in shp))
    consts = [wts["We0"], wts["be0"], wts["We1"], wts["be1"], mats["Rsf"],
              mats["Qd"], wts["Wo0"], wts["bo0"], wts["Wo1"], wts["bo1"],
              wts["Wo2"], wts["bo2"]]
    return pl.pallas_call(
        _node_body,
        grid=(n // bn,),
        in_specs=[
            pl.BlockSpec((bn, 128), lambda i: (i, 0)),
            pl.BlockSpec((8, 32), lambda i: (0, 0)),
        ] + [full(c.shape) for c in consts],
        out_specs=pl.BlockSpec((bn, 8, 128), lambda i: (i, 0, 0)),
        out_shape=jax.ShapeDtypeStruct((n, 8, 128), jnp.float32),
    )(parts, efeat, *consts)


def kernel(h, h_full, z, mask, e_feat, att_src, att_dst, att_dist, att_vec,
           z_emb_table, Wvw0, bvw0, Wvw1, bvw1, Wg0, bg0, Wg1, bg1,
           We0, be0, We1, be1, Wo0, bo0, Wo1, bo1, Wo2, bo2):
    f32 = jnp.float32
    B, N, hd = h.shape
    E = att_src.shape[0]
    nE = e_feat.shape[0]

    h2 = h.reshape(N, hd).astype(f32)
    hf2 = h_full.reshape(N, 40).astype(f32)
    zcol = z.reshape(N, 1).astype(f32)
    tab_pad = jnp.pad(z_emb_table.astype(f32),
                      ((0, 128 - z_emb_table.shape[0]), (0, 0)))

    idx_src = att_src.reshape(E).astype(jnp.int32)
    idx_dst = att_dst.reshape(E).astype(jnp.int32)
    esc = jnp.concatenate([
        att_dist.reshape(E, 1).astype(f32),
        att_vec.reshape(E, 3).astype(f32),
        idx_src.reshape(E, 1).astype(f32),
        idx_dst.reshape(E, 1).astype(f32),
        jnp.zeros((E, 2), f32),
    ], axis=1)

    wts = dict(
        Wvw0a=Wvw0[0:32], wv0self=Wvw0[32:33], Wvw0c=Wvw0[33:49],
        bvw0=bvw0.reshape(1, -1), Wvw1=Wvw1, bvw1=bvw1.reshape(1, -1),
        Wg0a=Wg0[0:128], Wg0b=Wg0[128:256], Wg0c=Wg0[256:272],
        wg0self=Wg0[272:273], bg0=bg0.reshape(1, -1),
        Wg1=Wg1, bg1=bg1.reshape(1, -1),
        We0=We0, be0=be0.reshape(1, -1), We1=We1, be1=be1.reshape(1, -1),
        Wo0=Wo0, bo0=bo0.reshape(1, -1), Wo1=Wo1, bo1=bo1.reshape(1, -1),
        Wo2=Wo2, bo2=bo2.reshape(1, -1),
    )
    mats = {k: jnp.asarray(v) for k, v in _MATS.items()}

    tdst = _prep_call(h2, hf2, zcol, tab_pad, bp=1000)
    dstg, srcg = _gather_call(tdst, h2, idx_dst, idx_src)
    ve = _edge_call(dstg, srcg, esc, wts, mats, bk=1000)

    npad = _NC * 2 * _RNG
    zeros_init = jnp.zeros((_RJ // _NS, 128), f32)
    parts = _scatter_call(ve, idx_src, zeros_init, npad)

    out = _node_call(parts, e_feat.astype(f32), wts, mats, N, bn=1000)
    return out.reshape(B, N, nE, 128)
